# Initial kernel scaffold; baseline (speedup 1.0000x reference)
import functools

import jax
import jax.numpy as jnp
from jax import lax
from jax.experimental import pallas as pl
from jax.experimental.pallas import tpu as pltpu
from jax.experimental.pallas import tpu_sc as plsc

N = 10000
E = 320000
D = 128
H = 256

NC = 2
NS = 16
NW = NC * NS

EPT = E // NW
CHUNK = 80
NCHUNK = EPT // CHUNK
ROWS_PT = N // NS


def _pre_body(x_ref, ws_ref, wr_ref, xs_ref, xr_ref):
    xb = x_ref[...]
    xs_ref[...] = jnp.dot(xb, ws_ref[...], preferred_element_type=jnp.float32)
    xr_ref[...] = jnp.dot(xb, wr_ref[...], preferred_element_type=jnp.float32)


def _pre_project(x, ws, wr):
    blk = 1000
    return pl.pallas_call(
        _pre_body,
        grid=(N // blk,),
        in_specs=[
            pl.BlockSpec((blk, D), lambda i: (i, 0)),
            pl.BlockSpec((D, H), lambda i: (0, 0)),
            pl.BlockSpec((D, H), lambda i: (0, 0)),
        ],
        out_specs=[
            pl.BlockSpec((blk, H), lambda i: (i, 0)),
            pl.BlockSpec((blk, H), lambda i: (i, 0)),
        ],
        out_shape=[
            jax.ShapeDtypeStruct((N, H), jnp.float32),
            jax.ShapeDtypeStruct((N, H), jnp.float32),
        ],
    )(x, ws, wr)


def _gather_body(xs_hbm, xr_hbm, sidx_hbm, ridx_hbm, out_hbm,
                 sidx_v, ridx_v, buf_a, buf_b, sem_a, sem_b):
    c = lax.axis_index("c")
    s = lax.axis_index("s")
    wid = s * NC + c
    tile_base = wid * EPT

    def chunk(j, carry):
        base = pl.multiple_of(tile_base + j * CHUNK, CHUNK)
        pltpu.sync_copy(sidx_hbm.at[pl.ds(base, CHUNK)], sidx_v)
        pltpu.sync_copy(ridx_hbm.at[pl.ds(base, CHUNK)], ridx_v)
        cp_a = pltpu.async_copy(xs_hbm.at[sidx_v], buf_a, sem_a)
        cp_b = pltpu.async_copy(xr_hbm.at[ridx_v], buf_b, sem_b)
        cp_a.wait()
        cp_b.wait()

        def add_row(r, carry2):
            for k in range(H // 16):
                sl = pl.ds(k * 16, 16)
                buf_a[r, sl] = buf_a[r, sl] + buf_b[r, sl]
            return carry2

        lax.fori_loop(0, CHUNK, add_row, 0, unroll=False)
        pltpu.sync_copy(buf_a, out_hbm.at[pl.ds(base, CHUNK)])
        return carry

    lax.fori_loop(0, NCHUNK, chunk, 0, unroll=False)


def _gather_sum(xs, xr, sidx, ridx):
    kfn = pl.kernel(
        _gather_body,
        out_type=jax.ShapeDtypeStruct((E, H), jnp.float32),
        mesh=plsc.VectorSubcoreMesh(core_axis_name="c", subcore_axis_name="s"),
        scratch_types=[
            pltpu.VMEM((CHUNK,), jnp.int32),
            pltpu.VMEM((CHUNK,), jnp.int32),
            pltpu.VMEM((CHUNK, H), jnp.float32),
            pltpu.VMEM((CHUNK, H), jnp.float32),
            pltpu.SemaphoreType.DMA,
            pltpu.SemaphoreType.DMA,
        ],
    )
    return kfn(xs, xr, sidx, ridx)


def _edge_body(ea_ref, g_ref, we_ref, be_ref, ge_ref, bne_ref, w2_ref, b2_ref,
               out_ref):
    hpre = (g_ref[...]
            + jnp.dot(ea_ref[...], we_ref[...],
                      preferred_element_type=jnp.float32)
            + be_ref[...])
    mu = jnp.mean(hpre, axis=-1, keepdims=True)
    var = jnp.mean((hpre - mu) ** 2, axis=-1, keepdims=True)
    hn = (hpre - mu) * lax.rsqrt(var + 1e-5) * ge_ref[...] + bne_ref[...]
    h = jnp.maximum(hn, 0.0)
    out_ref[...] = (jnp.dot(h, w2_ref[...], preferred_element_type=jnp.float32)
                    + b2_ref[...])


def _edge_mlp(edge_attr, gathered, we, be, ge, bne, w2, b2):
    blk = 1000
    return pl.pallas_call(
        _edge_body,
        grid=(E // blk,),
        in_specs=[
            pl.BlockSpec((blk, D), lambda i: (i, 0)),
            pl.BlockSpec((blk, H), lambda i: (i, 0)),
            pl.BlockSpec((D, H), lambda i: (0, 0)),
            pl.BlockSpec((1, H), lambda i: (0, 0)),
            pl.BlockSpec((1, H), lambda i: (0, 0)),
            pl.BlockSpec((1, H), lambda i: (0, 0)),
            pl.BlockSpec((H, D), lambda i: (0, 0)),
            pl.BlockSpec((1, D), lambda i: (0, 0)),
        ],
        out_specs=pl.BlockSpec((blk, D), lambda i: (i, 0)),
        out_shape=jax.ShapeDtypeStruct((E, D), jnp.float32),
    )(edge_attr, gathered, we, be, ge, bne, w2, b2)


def _scatter_body(upd_hbm, ridx_hbm, zeros_hbm, out_hbm,
                  idx_v, upd_v, acc_sh, sem):
    c = lax.axis_index("c")
    s = lax.axis_index("s")
    pltpu.sync_copy(zeros_hbm.at[pl.ds(s * ROWS_PT, ROWS_PT)],
                    acc_sh.at[pl.ds(s * ROWS_PT, ROWS_PT)])
    plsc.subcore_barrier()

    tile_base = (c * NS + s) * EPT

    def chunk(j, carry):
        base = pl.multiple_of(tile_base + j * CHUNK, CHUNK)
        pltpu.sync_copy(ridx_hbm.at[pl.ds(base, CHUNK)], idx_v)
        pltpu.sync_copy(upd_hbm.at[pl.ds(base, CHUNK)], upd_v)
        pltpu.sync_copy(upd_v, acc_sh.at[idx_v], add=True)
        return carry

    lax.fori_loop(0, NCHUNK, chunk, 0, unroll=False)
    plsc.subcore_barrier()
    pltpu.sync_copy(acc_sh.at[pl.ds(s * ROWS_PT, ROWS_PT)],
                    out_hbm.at[c].at[pl.ds(s * ROWS_PT, ROWS_PT)])


def _scatter_partials(upd, ridx, zeros):
    kfn = pl.kernel(
        _scatter_body,
        out_type=jax.ShapeDtypeStruct((NC, N, D), jnp.float32),
        mesh=plsc.VectorSubcoreMesh(core_axis_name="c", subcore_axis_name="s"),
        scratch_types=[
            pltpu.VMEM((CHUNK,), jnp.int32),
            pltpu.VMEM((CHUNK, D), jnp.float32),
            pltpu.VMEM_SHARED((N, D), jnp.float32),
            pltpu.SemaphoreType.DMA,
        ],
    )
    return kfn(upd, ridx, zeros)


def _node_body(x_ref, aa_ref, ab_ref, wx_ref, wa_ref, bn_ref, gn_ref,
               bnn_ref, w2_ref, b2_ref, out_ref):
    xb = x_ref[...]
    agg = aa_ref[...] + ab_ref[...]
    hpre = (jnp.dot(xb, wx_ref[...], preferred_element_type=jnp.float32)
            + jnp.dot(agg, wa_ref[...], preferred_element_type=jnp.float32)
            + bn_ref[...])
    mu = jnp.mean(hpre, axis=-1, keepdims=True)
    var = jnp.mean((hpre - mu) ** 2, axis=-1, keepdims=True)
    hn = (hpre - mu) * lax.rsqrt(var + 1e-5) * gn_ref[...] + bnn_ref[...]
    h = jnp.maximum(hn, 0.0)
    out_ref[...] = (xb
                    + jnp.dot(h, w2_ref[...],
                              preferred_element_type=jnp.float32)
                    + b2_ref[...])


def _node_mlp(x, agg_a, agg_b, wx, wa, bn, gn, bnn, w2, b2):
    blk = 1000
    return pl.pallas_call(
        _node_body,
        grid=(N // blk,),
        in_specs=[
            pl.BlockSpec((blk, D), lambda i: (i, 0)),
            pl.BlockSpec((blk, D), lambda i: (i, 0)),
            pl.BlockSpec((blk, D), lambda i: (i, 0)),
            pl.BlockSpec((D, H), lambda i: (0, 0)),
            pl.BlockSpec((D, H), lambda i: (0, 0)),
            pl.BlockSpec((1, H), lambda i: (0, 0)),
            pl.BlockSpec((1, H), lambda i: (0, 0)),
            pl.BlockSpec((1, H), lambda i: (0, 0)),
            pl.BlockSpec((H, D), lambda i: (0, 0)),
            pl.BlockSpec((1, D), lambda i: (0, 0)),
        ],
        out_specs=pl.BlockSpec((blk, D), lambda i: (i, 0)),
        out_shape=jax.ShapeDtypeStruct((N, D), jnp.float32),
    )(x, agg_a, agg_b, wx, wa, bn, gn, bnn, w2, b2)


def kernel(x, edge_index, edge_attr, We1, be1, ge1, bne1, We2, be2,
           Wn1, bn1, gn1, bnn1, Wn2, bn2):
    sidx = edge_index[0].astype(jnp.int32)
    ridx = edge_index[1].astype(jnp.int32)

    we_e = We1[:D]
    we_s = We1[D:2 * D]
    we_r = We1[2 * D:]
    wn_x = Wn1[:D]
    wn_a = Wn1[D:]

    xs, xr = _pre_project(x, we_s, we_r)
    gathered = _gather_sum(xs, xr, sidx, ridx)
    upd = _edge_mlp(edge_attr, gathered,
                    we_e, be1.reshape(1, H), ge1.reshape(1, H),
                    bne1.reshape(1, H), We2, be2.reshape(1, D))
    zeros = jnp.zeros((N, D), jnp.float32)
    partials = _scatter_partials(upd, ridx, zeros)
    out = _node_mlp(x, partials[0], partials[1],
                    wn_x, wn_a, bn1.reshape(1, H), gn1.reshape(1, H),
                    bnn1.reshape(1, H), Wn2, bn2.reshape(1, D))
    return out


# speedup vs baseline: 2.2742x; 2.2742x over previous
import functools

import jax
import jax.numpy as jnp
from jax import lax
from jax.experimental import pallas as pl
from jax.experimental.pallas import tpu as pltpu
from jax.experimental.pallas import tpu_sc as plsc

N = 10000
E = 320000
D = 128
H = 256

NC = 2
NS = 16
NW = NC * NS

EPT = E // NW
CHUNK = 80
NCHUNK = EPT // CHUNK
SLAB = 624
TAIL = N - SLAB * NS


def _pre_body(x_ref, ws_ref, wr_ref, xs_ref, xr_ref):
    xb = x_ref[...]
    xs_ref[...] = jnp.dot(xb, ws_ref[...], preferred_element_type=jnp.float32)
    xr_ref[...] = jnp.dot(xb, wr_ref[...], preferred_element_type=jnp.float32)


def _pre_project(x, ws, wr):
    blk = 1000
    return pl.pallas_call(
        _pre_body,
        grid=(N // blk,),
        in_specs=[
            pl.BlockSpec((blk, D), lambda i: (i, 0)),
            pl.BlockSpec((D, H), lambda i: (0, 0)),
            pl.BlockSpec((D, H), lambda i: (0, 0)),
        ],
        out_specs=[
            pl.BlockSpec((blk, H), lambda i: (i, 0)),
            pl.BlockSpec((blk, H), lambda i: (i, 0)),
        ],
        out_shape=[
            jax.ShapeDtypeStruct((N, H), jnp.float32),
            jax.ShapeDtypeStruct((N, H), jnp.float32),
        ],
    )(x, ws, wr)


def _gather_body(xs_hbm, xr_hbm, sidx_hbm, ridx_hbm, out_hbm,
                 sidx_v, ridx_v, buf_a, buf_b, sem_a, sem_b):
    c = lax.axis_index("c")
    s = lax.axis_index("s")
    wid = s * NC + c
    tile_base = wid * EPT

    def chunk(j, carry):
        base = pl.multiple_of(tile_base + j * CHUNK, CHUNK)
        pltpu.sync_copy(sidx_hbm.at[pl.ds(base, CHUNK)], sidx_v)
        pltpu.sync_copy(ridx_hbm.at[pl.ds(base, CHUNK)], ridx_v)
        cp_a = pltpu.async_copy(xs_hbm.at[sidx_v], buf_a, sem_a)
        cp_b = pltpu.async_copy(xr_hbm.at[ridx_v], buf_b, sem_b)
        cp_a.wait()
        cp_b.wait()

        def add_row(r, carry2):
            for k in range(H // 16):
                sl = pl.ds(k * 16, 16)
                buf_a[r, sl] = buf_a[r, sl] + buf_b[r, sl]
            return carry2

        lax.fori_loop(0, CHUNK, add_row, 0, unroll=False)
        pltpu.sync_copy(buf_a, out_hbm.at[pl.ds(base, CHUNK)])
        return carry

    lax.fori_loop(0, NCHUNK, chunk, 0, unroll=False)


def _gather_sum(xs, xr, sidx, ridx):
    kfn = pl.kernel(
        _gather_body,
        out_type=jax.ShapeDtypeStruct((E, H), jnp.float32),
        mesh=plsc.VectorSubcoreMesh(core_axis_name="c", subcore_axis_name="s"),
        scratch_types=[
            pltpu.VMEM((CHUNK,), jnp.int32),
            pltpu.VMEM((CHUNK,), jnp.int32),
            pltpu.VMEM((CHUNK, H), jnp.float32),
            pltpu.VMEM((CHUNK, H), jnp.float32),
            pltpu.SemaphoreType.DMA,
            pltpu.SemaphoreType.DMA,
        ],
    )
    return kfn(xs, xr, sidx, ridx)


def _edge_body(ea_ref, g_ref, we_ref, be_ref, ge_ref, bne_ref, w2_ref, b2_ref,
               out_ref):
    hpre = (g_ref[...]
            + jnp.dot(ea_ref[...], we_ref[...],
                      preferred_element_type=jnp.float32)
            + be_ref[...])
    mu = jnp.mean(hpre, axis=-1, keepdims=True)
    var = jnp.mean((hpre - mu) ** 2, axis=-1, keepdims=True)
    hn = (hpre - mu) * lax.rsqrt(var + 1e-5) * ge_ref[...] + bne_ref[...]
    h = jnp.maximum(hn, 0.0)
    out_ref[...] = (jnp.dot(h, w2_ref[...], preferred_element_type=jnp.float32)
                    + b2_ref[...])


def _edge_mlp(edge_attr, gathered, we, be, ge, bne, w2, b2):
    blk = 1000
    return pl.pallas_call(
        _edge_body,
        grid=(E // blk,),
        in_specs=[
            pl.BlockSpec((blk, D), lambda i: (i, 0)),
            pl.BlockSpec((blk, H), lambda i: (i, 0)),
            pl.BlockSpec((D, H), lambda i: (0, 0)),
            pl.BlockSpec((1, H), lambda i: (0, 0)),
            pl.BlockSpec((1, H), lambda i: (0, 0)),
            pl.BlockSpec((1, H), lambda i: (0, 0)),
            pl.BlockSpec((H, D), lambda i: (0, 0)),
            pl.BlockSpec((1, D), lambda i: (0, 0)),
        ],
        out_specs=pl.BlockSpec((blk, D), lambda i: (i, 0)),
        out_shape=jax.ShapeDtypeStruct((E, D), jnp.float32),
    )(edge_attr, gathered, we, be, ge, bne, w2, b2)


def _scatter_body(upd_hbm, ridx_hbm, zeros_hbm, out_hbm,
                  idx_v, upd_v, acc_sh, sem):
    c = lax.axis_index("c")
    s = lax.axis_index("s")
    slab = pl.multiple_of(s * SLAB, 8)
    pltpu.sync_copy(zeros_hbm.at[pl.ds(slab, SLAB)],
                    acc_sh.at[pl.ds(slab, SLAB)])

    @pl.when(s == 0)
    def _init_tail():
        pltpu.sync_copy(zeros_hbm.at[pl.ds(SLAB * NS, TAIL)],
                        acc_sh.at[pl.ds(SLAB * NS, TAIL)])

    plsc.subcore_barrier()

    tile_base = (c * NS + s) * EPT

    def chunk(j, carry):
        base = pl.multiple_of(tile_base + j * CHUNK, CHUNK)
        pltpu.sync_copy(ridx_hbm.at[pl.ds(base, CHUNK)], idx_v)
        pltpu.sync_copy(upd_hbm.at[pl.ds(base, CHUNK)], upd_v)
        pltpu.sync_copy(upd_v, acc_sh.at[idx_v], add=True)
        return carry

    lax.fori_loop(0, NCHUNK, chunk, 0, unroll=False)
    plsc.subcore_barrier()
    pltpu.sync_copy(acc_sh.at[pl.ds(slab, SLAB)],
                    out_hbm.at[c].at[pl.ds(slab, SLAB)])

    @pl.when(s == 0)
    def _out_tail():
        pltpu.sync_copy(acc_sh.at[pl.ds(SLAB * NS, TAIL)],
                        out_hbm.at[c].at[pl.ds(SLAB * NS, TAIL)])


def _scatter_partials(upd, ridx, zeros):
    kfn = pl.kernel(
        _scatter_body,
        out_type=jax.ShapeDtypeStruct((NC, N, D), jnp.float32),
        mesh=plsc.VectorSubcoreMesh(core_axis_name="c", subcore_axis_name="s"),
        scratch_types=[
            pltpu.VMEM((CHUNK,), jnp.int32),
            pltpu.VMEM((CHUNK, D), jnp.float32),
            pltpu.VMEM_SHARED((N, D), jnp.float32),
            pltpu.SemaphoreType.DMA,
        ],
    )
    return kfn(upd, ridx, zeros)


def _node_body(x_ref, aa_ref, ab_ref, wx_ref, wa_ref, bn_ref, gn_ref,
               bnn_ref, w2_ref, b2_ref, out_ref):
    xb = x_ref[...]
    agg = aa_ref[...] + ab_ref[...]
    hpre = (jnp.dot(xb, wx_ref[...], preferred_element_type=jnp.float32)
            + jnp.dot(agg, wa_ref[...], preferred_element_type=jnp.float32)
            + bn_ref[...])
    mu = jnp.mean(hpre, axis=-1, keepdims=True)
    var = jnp.mean((hpre - mu) ** 2, axis=-1, keepdims=True)
    hn = (hpre - mu) * lax.rsqrt(var + 1e-5) * gn_ref[...] + bnn_ref[...]
    h = jnp.maximum(hn, 0.0)
    out_ref[...] = (xb
                    + jnp.dot(h, w2_ref[...],
                              preferred_element_type=jnp.float32)
                    + b2_ref[...])


def _node_mlp(x, agg_a, agg_b, wx, wa, bn, gn, bnn, w2, b2):
    blk = 1000
    return pl.pallas_call(
        _node_body,
        grid=(N // blk,),
        in_specs=[
            pl.BlockSpec((blk, D), lambda i: (i, 0)),
            pl.BlockSpec((blk, D), lambda i: (i, 0)),
            pl.BlockSpec((blk, D), lambda i: (i, 0)),
            pl.BlockSpec((D, H), lambda i: (0, 0)),
            pl.BlockSpec((D, H), lambda i: (0, 0)),
            pl.BlockSpec((1, H), lambda i: (0, 0)),
            pl.BlockSpec((1, H), lambda i: (0, 0)),
            pl.BlockSpec((1, H), lambda i: (0, 0)),
            pl.BlockSpec((H, D), lambda i: (0, 0)),
            pl.BlockSpec((1, D), lambda i: (0, 0)),
        ],
        out_specs=pl.BlockSpec((blk, D), lambda i: (i, 0)),
        out_shape=jax.ShapeDtypeStruct((N, D), jnp.float32),
    )(x, agg_a, agg_b, wx, wa, bn, gn, bnn, w2, b2)


def kernel(x, edge_index, edge_attr, We1, be1, ge1, bne1, We2, be2,
           Wn1, bn1, gn1, bnn1, Wn2, bn2):
    sidx = edge_index[0].astype(jnp.int32)
    ridx = edge_index[1].astype(jnp.int32)

    we_e = We1[:D]
    we_s = We1[D:2 * D]
    we_r = We1[2 * D:]
    wn_x = Wn1[:D]
    wn_a = Wn1[D:]

    xs, xr = _pre_project(x, we_s, we_r)
    gathered = _gather_sum(xs, xr, sidx, ridx)
    upd = _edge_mlp(edge_attr, gathered,
                    we_e, be1.reshape(1, H), ge1.reshape(1, H),
                    bne1.reshape(1, H), We2, be2.reshape(1, D))
    zeros = jnp.zeros((N, D), jnp.float32)
    partials = _scatter_partials(upd, ridx, zeros)
    out = _node_mlp(x, partials[0], partials[1],
                    wn_x, wn_a, bn1.reshape(1, H), gn1.reshape(1, H),
                    bnn1.reshape(1, H), Wn2, bn2.reshape(1, D))
    return out


# speedup vs baseline: 2.6880x; 1.1820x over previous
import functools

import jax
import jax.numpy as jnp
from jax import lax
from jax.experimental import pallas as pl
from jax.experimental.pallas import tpu as pltpu
from jax.experimental.pallas import tpu_sc as plsc

N = 10000
E = 320000
D = 128
H = 256

NC = 2
NS = 16
NW = NC * NS

EPT = E // NW
CHUNK = 40
NCHUNK = EPT // CHUNK
SBLK = 40
SCPB = SBLK // CHUNK
NSBLK = EPT // SBLK
SLAB = 624
TAIL = N - SLAB * NS


def _pre_body(x_ref, ws_ref, wr_ref, xs_ref, xr_ref):
    xb = x_ref[...]
    xs_ref[...] = jnp.dot(xb, ws_ref[...], preferred_element_type=jnp.float32)
    xr_ref[...] = jnp.dot(xb, wr_ref[...], preferred_element_type=jnp.float32)


def _pre_project(x, ws, wr):
    blk = 1000
    return pl.pallas_call(
        _pre_body,
        grid=(N // blk,),
        in_specs=[
            pl.BlockSpec((blk, D), lambda i: (i, 0)),
            pl.BlockSpec((D, H), lambda i: (0, 0)),
            pl.BlockSpec((D, H), lambda i: (0, 0)),
        ],
        out_specs=[
            pl.BlockSpec((blk, H), lambda i: (i, 0)),
            pl.BlockSpec((blk, H), lambda i: (i, 0)),
        ],
        out_shape=[
            jax.ShapeDtypeStruct((N, H), jnp.float32),
            jax.ShapeDtypeStruct((N, H), jnp.float32),
        ],
    )(x, ws, wr)


def _gather_body(xs_hbm, xr_hbm, sidx_hbm, ridx_hbm, out_hbm,
                 sidx_all, ridx_all,
                 buf_a0, buf_b0, out0, buf_a1, buf_b1, out1,
                 sem_a0, sem_b0, sem_o0, sem_a1, sem_b1, sem_o1):
    c = lax.axis_index("c")
    s = lax.axis_index("s")
    wid = s * NC + c
    tile_base = wid * EPT
    pltpu.sync_copy(sidx_hbm.at[wid], sidx_all)
    pltpu.sync_copy(ridx_hbm.at[wid], ridx_all)

    slots = ((buf_a0, buf_b0, out0, sem_a0, sem_b0, sem_o0),
             (buf_a1, buf_b1, out1, sem_a1, sem_b1, sem_o1))

    def issue(q, sl):
        ba, bb, _, sa, sb, _ = slots[sl]
        pltpu.async_copy(xs_hbm.at[sidx_all.at[q]], ba, sa)
        pltpu.async_copy(xr_hbm.at[ridx_all.at[q]], bb, sb)

    def process(q, sl, first_pair, last):
        ba, bb, ob, sa, sb, so = slots[sl]
        pltpu.make_async_copy(xs_hbm.at[sidx_all.at[0]], ba, sa).wait()
        pltpu.make_async_copy(xr_hbm.at[ridx_all.at[0]], bb, sb).wait()

        def add_row(r, carry):
            for k in range(H // 16):
                slc = pl.ds(k * 16, 16)
                ob[r, slc] = ba[r, slc] + bb[r, slc]
            return carry

        lax.fori_loop(0, CHUNK, add_row, 0, unroll=4)

        @pl.when(q + 2 < NCHUNK)
        def _issue_next():
            issue(q + 2, sl)

        @pl.when(jnp.logical_not(first_pair))
        def _drain_prev_out():
            pltpu.make_async_copy(ob, out_hbm.at[pl.ds(0, CHUNK)], so).wait()

        base = pl.multiple_of(tile_base + q * CHUNK, CHUNK)
        pltpu.async_copy(ob, out_hbm.at[pl.ds(base, CHUNK)], so)

    issue(0, 0)
    issue(1, 1)

    def pair(t, carry):
        process(2 * t, 0, t == 0, False)
        process(2 * t + 1, 1, t == 0, False)
        return carry

    lax.fori_loop(0, NCHUNK // 2, pair, 0, unroll=False)
    pltpu.make_async_copy(out0, out_hbm.at[pl.ds(0, CHUNK)], sem_o0).wait()
    pltpu.make_async_copy(out1, out_hbm.at[pl.ds(0, CHUNK)], sem_o1).wait()


def _gather_sum(xs, xr, sidx3, ridx3):
    kfn = pl.kernel(
        _gather_body,
        out_type=jax.ShapeDtypeStruct((E, H), jnp.float32),
        mesh=plsc.VectorSubcoreMesh(core_axis_name="c", subcore_axis_name="s"),
        scratch_types=[
            pltpu.VMEM((NCHUNK, CHUNK), jnp.int32),
            pltpu.VMEM((NCHUNK, CHUNK), jnp.int32),
            pltpu.VMEM((CHUNK, H), jnp.float32),
            pltpu.VMEM((CHUNK, H), jnp.float32),
            pltpu.VMEM((CHUNK, H), jnp.float32),
            pltpu.VMEM((CHUNK, H), jnp.float32),
            pltpu.VMEM((CHUNK, H), jnp.float32),
            pltpu.VMEM((CHUNK, H), jnp.float32),
            pltpu.SemaphoreType.DMA,
            pltpu.SemaphoreType.DMA,
            pltpu.SemaphoreType.DMA,
            pltpu.SemaphoreType.DMA,
            pltpu.SemaphoreType.DMA,
            pltpu.SemaphoreType.DMA,
        ],
    )
    return kfn(xs, xr, sidx3, ridx3)


def _edge_body(ea_ref, g_ref, we_ref, be_ref, ge_ref, bne_ref, w2_ref, b2_ref,
               out_ref):
    hpre = (g_ref[...]
            + jnp.dot(ea_ref[...], we_ref[...],
                      preferred_element_type=jnp.float32)
            + be_ref[...])
    mu = jnp.mean(hpre, axis=-1, keepdims=True)
    var = jnp.mean((hpre - mu) ** 2, axis=-1, keepdims=True)
    hn = (hpre - mu) * lax.rsqrt(var + 1e-5) * ge_ref[...] + bne_ref[...]
    h = jnp.maximum(hn, 0.0)
    out_ref[...] = (jnp.dot(h, w2_ref[...], preferred_element_type=jnp.float32)
                    + b2_ref[...])


def _edge_mlp(edge_attr, gathered, we, be, ge, bne, w2, b2):
    blk = 1000
    return pl.pallas_call(
        _edge_body,
        grid=(E // blk,),
        in_specs=[
            pl.BlockSpec((blk, D), lambda i: (i, 0)),
            pl.BlockSpec((blk, H), lambda i: (i, 0)),
            pl.BlockSpec((D, H), lambda i: (0, 0)),
            pl.BlockSpec((1, H), lambda i: (0, 0)),
            pl.BlockSpec((1, H), lambda i: (0, 0)),
            pl.BlockSpec((1, H), lambda i: (0, 0)),
            pl.BlockSpec((H, D), lambda i: (0, 0)),
            pl.BlockSpec((1, D), lambda i: (0, 0)),
        ],
        out_specs=pl.BlockSpec((blk, D), lambda i: (i, 0)),
        out_shape=jax.ShapeDtypeStruct((E, D), jnp.float32),
    )(edge_attr, gathered, we, be, ge, bne, w2, b2)


def _scatter_body(upd_hbm, ridx_hbm, zeros_hbm, out_hbm,
                  idx_all, upd0, upd1, acc_sh, sem_u0, sem_u1):
    c = lax.axis_index("c")
    s = lax.axis_index("s")
    wid = c * NS + s
    tile_base = wid * EPT
    pltpu.sync_copy(ridx_hbm.at[wid], idx_all)
    slab = pl.multiple_of(s * SLAB, 8)
    pltpu.sync_copy(zeros_hbm.at[pl.ds(slab, SLAB)],
                    acc_sh.at[pl.ds(slab, SLAB)])

    @pl.when(s == 0)
    def _init_tail():
        pltpu.sync_copy(zeros_hbm.at[pl.ds(SLAB * NS, TAIL)],
                        acc_sh.at[pl.ds(SLAB * NS, TAIL)])

    plsc.subcore_barrier()

    slots = ((upd0, sem_u0), (upd1, sem_u1))

    def issue(b, sl):
        buf, sem = slots[sl]
        base = pl.multiple_of(tile_base + b * SBLK, CHUNK)
        pltpu.async_copy(upd_hbm.at[pl.ds(base, SBLK)], buf, sem)

    def process(b, sl):
        buf, sem = slots[sl]
        pltpu.make_async_copy(upd_hbm.at[pl.ds(0, SBLK)], buf, sem).wait()
        for k in range(SCPB):
            pltpu.sync_copy(buf.at[pl.ds(k * CHUNK, CHUNK)],
                            acc_sh.at[idx_all.at[b * SCPB + k]], add=True)

        @pl.when(b + 2 < NSBLK)
        def _issue_next():
            issue(b + 2, sl)

    issue(0, 0)
    issue(1, 1)

    def pair(t, carry):
        process(2 * t, 0)
        process(2 * t + 1, 1)
        return carry

    lax.fori_loop(0, NSBLK // 2, pair, 0, unroll=False)

    plsc.subcore_barrier()
    pltpu.sync_copy(acc_sh.at[pl.ds(slab, SLAB)],
                    out_hbm.at[c].at[pl.ds(slab, SLAB)])

    @pl.when(s == 0)
    def _out_tail():
        pltpu.sync_copy(acc_sh.at[pl.ds(SLAB * NS, TAIL)],
                        out_hbm.at[c].at[pl.ds(SLAB * NS, TAIL)])


def _scatter_partials(upd, ridx3, zeros):
    kfn = pl.kernel(
        _scatter_body,
        out_type=jax.ShapeDtypeStruct((NC, N, D), jnp.float32),
        mesh=plsc.VectorSubcoreMesh(core_axis_name="c", subcore_axis_name="s"),
        scratch_types=[
            pltpu.VMEM((NCHUNK, CHUNK), jnp.int32),
            pltpu.VMEM((SBLK, D), jnp.float32),
            pltpu.VMEM((SBLK, D), jnp.float32),
            pltpu.VMEM_SHARED((N, D), jnp.float32),
            pltpu.SemaphoreType.DMA,
            pltpu.SemaphoreType.DMA,
        ],
    )
    return kfn(upd, ridx3, zeros)


def _node_body(x_ref, aa_ref, ab_ref, wx_ref, wa_ref, bn_ref, gn_ref,
               bnn_ref, w2_ref, b2_ref, out_ref):
    xb = x_ref[...]
    agg = aa_ref[...] + ab_ref[...]
    hpre = (jnp.dot(xb, wx_ref[...], preferred_element_type=jnp.float32)
            + jnp.dot(agg, wa_ref[...], preferred_element_type=jnp.float32)
            + bn_ref[...])
    mu = jnp.mean(hpre, axis=-1, keepdims=True)
    var = jnp.mean((hpre - mu) ** 2, axis=-1, keepdims=True)
    hn = (hpre - mu) * lax.rsqrt(var + 1e-5) * gn_ref[...] + bnn_ref[...]
    h = jnp.maximum(hn, 0.0)
    out_ref[...] = (xb
                    + jnp.dot(h, w2_ref[...],
                              preferred_element_type=jnp.float32)
                    + b2_ref[...])


def _node_mlp(x, agg_a, agg_b, wx, wa, bn, gn, bnn, w2, b2):
    blk = 1000
    return pl.pallas_call(
        _node_body,
        grid=(N // blk,),
        in_specs=[
            pl.BlockSpec((blk, D), lambda i: (i, 0)),
            pl.BlockSpec((blk, D), lambda i: (i, 0)),
            pl.BlockSpec((blk, D), lambda i: (i, 0)),
            pl.BlockSpec((D, H), lambda i: (0, 0)),
            pl.BlockSpec((D, H), lambda i: (0, 0)),
            pl.BlockSpec((1, H), lambda i: (0, 0)),
            pl.BlockSpec((1, H), lambda i: (0, 0)),
            pl.BlockSpec((1, H), lambda i: (0, 0)),
            pl.BlockSpec((H, D), lambda i: (0, 0)),
            pl.BlockSpec((1, D), lambda i: (0, 0)),
        ],
        out_specs=pl.BlockSpec((blk, D), lambda i: (i, 0)),
        out_shape=jax.ShapeDtypeStruct((N, D), jnp.float32),
    )(x, agg_a, agg_b, wx, wa, bn, gn, bnn, w2, b2)


def kernel(x, edge_index, edge_attr, We1, be1, ge1, bne1, We2, be2,
           Wn1, bn1, gn1, bnn1, Wn2, bn2):
    sidx = edge_index[0].astype(jnp.int32).reshape(NW, NCHUNK, CHUNK)
    ridx = edge_index[1].astype(jnp.int32).reshape(NW, NCHUNK, CHUNK)

    we_e = We1[:D]
    we_s = We1[D:2 * D]
    we_r = We1[2 * D:]
    wn_x = Wn1[:D]
    wn_a = Wn1[D:]

    xs, xr = _pre_project(x, we_s, we_r)
    gathered = _gather_sum(xs, xr, sidx, ridx)
    upd = _edge_mlp(edge_attr, gathered,
                    we_e, be1.reshape(1, H), ge1.reshape(1, H),
                    bne1.reshape(1, H), We2, be2.reshape(1, D))
    zeros = jnp.zeros((N, D), jnp.float32)
    partials = _scatter_partials(upd, ridx, zeros)
    out = _node_mlp(x, partials[0], partials[1],
                    wn_x, wn_a, bn1.reshape(1, H), gn1.reshape(1, H),
                    bnn1.reshape(1, H), Wn2, bn2.reshape(1, D))
    return out


# speedup vs baseline: 3.2666x; 1.2153x over previous
import functools

import jax
import jax.numpy as jnp
from jax import lax
from jax.experimental import pallas as pl
from jax.experimental.pallas import tpu as pltpu
from jax.experimental.pallas import tpu_sc as plsc

N = 10000
E = 320000
D = 128
H = 256

NC = 2
NS = 16
NW = NC * NS

EPT = E // NW
CHUNK = 40
NCHUNK = EPT // CHUNK
SBLK = 40
SCPB = SBLK // CHUNK
NSBLK = EPT // SBLK
SLAB = 624
TAIL = N - SLAB * NS


def _pre_body(x_ref, ws_ref, wr_ref, xs_ref, xr_ref):
    xb = x_ref[...]
    xs_ref[...] = jnp.dot(xb, ws_ref[...], preferred_element_type=jnp.float32)
    xr_ref[...] = jnp.dot(xb, wr_ref[...], preferred_element_type=jnp.float32)


def _pre_project(x, ws, wr):
    blk = 1000
    return pl.pallas_call(
        _pre_body,
        grid=(N // blk,),
        in_specs=[
            pl.BlockSpec((blk, D), lambda i: (i, 0)),
            pl.BlockSpec((D, H), lambda i: (0, 0)),
            pl.BlockSpec((D, H), lambda i: (0, 0)),
        ],
        out_specs=[
            pl.BlockSpec((blk, H), lambda i: (i, 0)),
            pl.BlockSpec((blk, H), lambda i: (i, 0)),
        ],
        out_shape=[
            jax.ShapeDtypeStruct((N, H), jnp.float32),
            jax.ShapeDtypeStruct((N, H), jnp.float32),
        ],
    )(x, ws, wr)


def _gather_body(xs_hbm, xr_hbm, sidx_hbm, ridx_hbm, out_hbm,
                 sidx_all, ridx_all,
                 buf_a0, buf_b0, out0, buf_a1, buf_b1, out1,
                 sem_a0, sem_b0, sem_o0, sem_a1, sem_b1, sem_o1):
    c = lax.axis_index("c")
    s = lax.axis_index("s")
    wid = s * NC + c
    tile_base = wid * EPT
    pltpu.sync_copy(sidx_hbm.at[wid], sidx_all)
    pltpu.sync_copy(ridx_hbm.at[wid], ridx_all)

    slots = ((buf_a0, buf_b0, out0, sem_a0, sem_b0, sem_o0),
             (buf_a1, buf_b1, out1, sem_a1, sem_b1, sem_o1))

    def issue(q, sl):
        ba, bb, _, sa, sb, _ = slots[sl]
        pltpu.async_copy(xs_hbm.at[sidx_all.at[q]], ba, sa)
        pltpu.async_copy(xr_hbm.at[ridx_all.at[q]], bb, sb)

    def process(q, sl, first_pair, last):
        ba, bb, ob, sa, sb, so = slots[sl]
        pltpu.make_async_copy(xs_hbm.at[sidx_all.at[0]], ba, sa).wait()
        pltpu.make_async_copy(xr_hbm.at[ridx_all.at[0]], bb, sb).wait()

        def add_row(r, carry):
            for k in range(H // 16):
                slc = pl.ds(k * 16, 16)
                ob[r, slc] = ba[r, slc] + bb[r, slc]
            return carry

        lax.fori_loop(0, CHUNK, add_row, 0, unroll=8)

        @pl.when(q + 2 < NCHUNK)
        def _issue_next():
            issue(q + 2, sl)

        @pl.when(jnp.logical_not(first_pair))
        def _drain_prev_out():
            pltpu.make_async_copy(ob, out_hbm.at[pl.ds(0, CHUNK)], so).wait()

        base = pl.multiple_of(tile_base + q * CHUNK, CHUNK)
        pltpu.async_copy(ob, out_hbm.at[pl.ds(base, CHUNK)], so)

    issue(0, 0)
    issue(1, 1)

    def pair(t, carry):
        process(2 * t, 0, t == 0, False)
        process(2 * t + 1, 1, t == 0, False)
        return carry

    lax.fori_loop(0, NCHUNK // 2, pair, 0, unroll=False)
    pltpu.make_async_copy(out0, out_hbm.at[pl.ds(0, CHUNK)], sem_o0).wait()
    pltpu.make_async_copy(out1, out_hbm.at[pl.ds(0, CHUNK)], sem_o1).wait()


def _gather_sum(xs, xr, sidx3, ridx3):
    kfn = pl.kernel(
        _gather_body,
        out_type=jax.ShapeDtypeStruct((E, H), jnp.float32),
        mesh=plsc.VectorSubcoreMesh(core_axis_name="c", subcore_axis_name="s"),
        scratch_types=[
            pltpu.VMEM((NCHUNK, CHUNK), jnp.int32),
            pltpu.VMEM((NCHUNK, CHUNK), jnp.int32),
            pltpu.VMEM((CHUNK, H), jnp.float32),
            pltpu.VMEM((CHUNK, H), jnp.float32),
            pltpu.VMEM((CHUNK, H), jnp.float32),
            pltpu.VMEM((CHUNK, H), jnp.float32),
            pltpu.VMEM((CHUNK, H), jnp.float32),
            pltpu.VMEM((CHUNK, H), jnp.float32),
            pltpu.SemaphoreType.DMA,
            pltpu.SemaphoreType.DMA,
            pltpu.SemaphoreType.DMA,
            pltpu.SemaphoreType.DMA,
            pltpu.SemaphoreType.DMA,
            pltpu.SemaphoreType.DMA,
        ],
    )
    return kfn(xs, xr, sidx3, ridx3)


def _edge_body(ea_ref, g_ref, we_ref, be_ref, ge_ref, bne_ref, w2_ref, b2_ref,
               out_ref):
    hpre = (g_ref[...]
            + jnp.dot(ea_ref[...], we_ref[...],
                      preferred_element_type=jnp.float32)
            + be_ref[...])
    mu = jnp.mean(hpre, axis=-1, keepdims=True)
    var = jnp.mean((hpre - mu) ** 2, axis=-1, keepdims=True)
    hn = (hpre - mu) * lax.rsqrt(var + 1e-5) * ge_ref[...] + bne_ref[...]
    h = jnp.maximum(hn, 0.0)
    out_ref[...] = (jnp.dot(h, w2_ref[...], preferred_element_type=jnp.float32)
                    + b2_ref[...])


def _edge_mlp(edge_attr, gathered, we, be, ge, bne, w2, b2):
    blk = 1000
    return pl.pallas_call(
        _edge_body,
        grid=(E // blk,),
        in_specs=[
            pl.BlockSpec((blk, D), lambda i: (i, 0)),
            pl.BlockSpec((blk, H), lambda i: (i, 0)),
            pl.BlockSpec((D, H), lambda i: (0, 0)),
            pl.BlockSpec((1, H), lambda i: (0, 0)),
            pl.BlockSpec((1, H), lambda i: (0, 0)),
            pl.BlockSpec((1, H), lambda i: (0, 0)),
            pl.BlockSpec((H, D), lambda i: (0, 0)),
            pl.BlockSpec((1, D), lambda i: (0, 0)),
        ],
        out_specs=pl.BlockSpec((blk, D), lambda i: (i, 0)),
        out_shape=jax.ShapeDtypeStruct((E, D), jnp.float32),
    )(edge_attr, gathered, we, be, ge, bne, w2, b2)


def _scatter_body(upd_hbm, ridx_hbm, zeros_hbm, out_hbm,
                  idx_all, upd0, upd1, acc_sh, sem_u0, sem_u1):
    c = lax.axis_index("c")
    s = lax.axis_index("s")
    wid = c * NS + s
    tile_base = wid * EPT
    pltpu.sync_copy(ridx_hbm.at[wid], idx_all)
    slab = pl.multiple_of(s * SLAB, 8)
    pltpu.sync_copy(zeros_hbm.at[pl.ds(slab, SLAB)],
                    acc_sh.at[pl.ds(slab, SLAB)])

    @pl.when(s == 0)
    def _init_tail():
        pltpu.sync_copy(zeros_hbm.at[pl.ds(SLAB * NS, TAIL)],
                        acc_sh.at[pl.ds(SLAB * NS, TAIL)])

    plsc.subcore_barrier()

    slots = ((upd0, sem_u0), (upd1, sem_u1))

    def issue(b, sl):
        buf, sem = slots[sl]
        base = pl.multiple_of(tile_base + b * SBLK, CHUNK)
        pltpu.async_copy(upd_hbm.at[pl.ds(base, SBLK)], buf, sem)

    def process(b, sl):
        buf, sem = slots[sl]
        pltpu.make_async_copy(upd_hbm.at[pl.ds(0, SBLK)], buf, sem).wait()
        for k in range(SCPB):
            pltpu.sync_copy(buf.at[pl.ds(k * CHUNK, CHUNK)],
                            acc_sh.at[idx_all.at[b * SCPB + k]], add=True)

        @pl.when(b + 2 < NSBLK)
        def _issue_next():
            issue(b + 2, sl)

    issue(0, 0)
    issue(1, 1)

    def pair(t, carry):
        process(2 * t, 0)
        process(2 * t + 1, 1)
        return carry

    lax.fori_loop(0, NSBLK // 2, pair, 0, unroll=False)

    plsc.subcore_barrier()
    pltpu.sync_copy(acc_sh.at[pl.ds(slab, SLAB)],
                    out_hbm.at[c].at[pl.ds(slab, SLAB)])

    @pl.when(s == 0)
    def _out_tail():
        pltpu.sync_copy(acc_sh.at[pl.ds(SLAB * NS, TAIL)],
                        out_hbm.at[c].at[pl.ds(SLAB * NS, TAIL)])


def _scatter_partials(upd, ridx3, zeros):
    kfn = pl.kernel(
        _scatter_body,
        out_type=jax.ShapeDtypeStruct((NC, N, D), jnp.float32),
        mesh=plsc.VectorSubcoreMesh(core_axis_name="c", subcore_axis_name="s"),
        scratch_types=[
            pltpu.VMEM((NCHUNK, CHUNK), jnp.int32),
            pltpu.VMEM((SBLK, D), jnp.float32),
            pltpu.VMEM((SBLK, D), jnp.float32),
            pltpu.VMEM_SHARED((N, D), jnp.float32),
            pltpu.SemaphoreType.DMA,
            pltpu.SemaphoreType.DMA,
        ],
    )
    return kfn(upd, ridx3, zeros)


def _node_body(x_ref, aa_ref, ab_ref, wx_ref, wa_ref, bn_ref, gn_ref,
               bnn_ref, w2_ref, b2_ref, out_ref):
    xb = x_ref[...]
    agg = aa_ref[...] + ab_ref[...]
    hpre = (jnp.dot(xb, wx_ref[...], preferred_element_type=jnp.float32)
            + jnp.dot(agg, wa_ref[...], preferred_element_type=jnp.float32)
            + bn_ref[...])
    mu = jnp.mean(hpre, axis=-1, keepdims=True)
    var = jnp.mean((hpre - mu) ** 2, axis=-1, keepdims=True)
    hn = (hpre - mu) * lax.rsqrt(var + 1e-5) * gn_ref[...] + bnn_ref[...]
    h = jnp.maximum(hn, 0.0)
    out_ref[...] = (xb
                    + jnp.dot(h, w2_ref[...],
                              preferred_element_type=jnp.float32)
                    + b2_ref[...])


def _node_mlp(x, agg_a, agg_b, wx, wa, bn, gn, bnn, w2, b2):
    blk = 1000
    return pl.pallas_call(
        _node_body,
        grid=(N // blk,),
        in_specs=[
            pl.BlockSpec((blk, D), lambda i: (i, 0)),
            pl.BlockSpec((blk, D), lambda i: (i, 0)),
            pl.BlockSpec((blk, D), lambda i: (i, 0)),
            pl.BlockSpec((D, H), lambda i: (0, 0)),
            pl.BlockSpec((D, H), lambda i: (0, 0)),
            pl.BlockSpec((1, H), lambda i: (0, 0)),
            pl.BlockSpec((1, H), lambda i: (0, 0)),
            pl.BlockSpec((1, H), lambda i: (0, 0)),
            pl.BlockSpec((H, D), lambda i: (0, 0)),
            pl.BlockSpec((1, D), lambda i: (0, 0)),
        ],
        out_specs=pl.BlockSpec((blk, D), lambda i: (i, 0)),
        out_shape=jax.ShapeDtypeStruct((N, D), jnp.float32),
    )(x, agg_a, agg_b, wx, wa, bn, gn, bnn, w2, b2)


def kernel(x, edge_index, edge_attr, We1, be1, ge1, bne1, We2, be2,
           Wn1, bn1, gn1, bnn1, Wn2, bn2):
    sidx = edge_index[0].astype(jnp.int32).reshape(NW, NCHUNK, CHUNK)
    ridx = edge_index[1].astype(jnp.int32).reshape(NW, NCHUNK, CHUNK)

    we_e = We1[:D]
    we_s = We1[D:2 * D]
    we_r = We1[2 * D:]
    wn_x = Wn1[:D]
    wn_a = Wn1[D:]

    xs, xr = _pre_project(x, we_s, we_r)
    gathered = _gather_sum(xs, xr, sidx, ridx)
    upd = _edge_mlp(edge_attr, gathered,
                    we_e, be1.reshape(1, H), ge1.reshape(1, H),
                    bne1.reshape(1, H), We2, be2.reshape(1, D))
    zeros = jnp.zeros((N, D), jnp.float32)
    partials = _scatter_partials(upd, ridx, zeros)
    out = _node_mlp(x, partials[0], partials[1],
                    wn_x, wn_a, bn1.reshape(1, H), gn1.reshape(1, H),
                    bnn1.reshape(1, H), Wn2, bn2.reshape(1, D))
    return out
